# baseline (speedup 1.0000x reference)
import functools

import jax
import jax.numpy as jnp
from jax import lax
from jax.experimental import pallas as pl
from jax.experimental.pallas import tpu as pltpu
from jax.experimental.pallas import tpu_sc as plsc

VOCAB = 1000000
MAXLEN = 200
EMBED = 16
BATCH = 4096

NUM_CORES = 2
NUM_SUBCORES = 16
NW = NUM_CORES * NUM_SUBCORES
SEQ_PER_W = BATCH // NW
CHUNK_SEQ = 16
ROWS_PER_CHUNK = CHUNK_SEQ * MAXLEN
GATHER = 128
N_GATHERS = ROWS_PER_CHUNK // GATHER
N_CHUNKS = SEQ_PER_W // CHUNK_SEQ
IDX_ROWS_PER_W = SEQ_PER_W * MAXLEN // GATHER


def _embed_kernel(x_hbm, tab_hbm, pos_hbm, out_hbm,
                  idx_a, idx_b, rows_a, rows_b, pos_v,
                  sem_i0, sem_i1, sem_g0, sem_g1, sem_o0, sem_o1):
    wid = lax.axis_index("s") * NUM_CORES + lax.axis_index("c")
    idx_bufs = (idx_a, idx_b)
    rows_bufs = (rows_a, rows_b)
    sem_i = (sem_i0, sem_i1)
    sem_g = (sem_g0, sem_g1)
    sem_o = (sem_o0, sem_o1)

    pltpu.sync_copy(pos_hbm, pos_v)
    base_row = wid * (SEQ_PER_W * MAXLEN)
    idx_base = wid * (SEQ_PER_W * MAXLEN)

    def fire_idx(c):
        b = c % 2
        return pltpu.async_copy(
            x_hbm.at[pl.ds(idx_base + c * ROWS_PER_CHUNK, ROWS_PER_CHUNK)],
            idx_bufs[b], sem_i[b])

    def fire_gathers(c):
        b = c % 2
        return [
            pltpu.async_copy(
                tab_hbm.at[idx_bufs[b]],
                rows_bufs[b],
                sem_g[b])
        ]

    def pos_add(b):
        def pos_body(p, carry):
            pv = pos_v[p, :]
            for s in range(CHUNK_SEQ):
                r = s * MAXLEN + p
                rows_bufs[b][r, :] = rows_bufs[b][r, :] + pv
            return carry

        lax.fori_loop(0, MAXLEN, pos_body, 0)

    idx_cp = {0: fire_idx(0)}
    idx_cp[0].wait()
    g_cps = {0: fire_gathers(0)}
    if N_CHUNKS > 1:
        idx_cp[1] = fire_idx(1)

    out_cp = {}
    for c in range(N_CHUNKS):
        b = c % 2
        if c + 1 < N_CHUNKS:
            if c - 1 >= 0:
                out_cp[c - 1].wait()
            idx_cp[c + 1].wait()
            g_cps[c + 1] = fire_gathers(c + 1)
        for cp in g_cps[c]:
            cp.wait()
        if c + 2 < N_CHUNKS:
            idx_cp[c + 2] = fire_idx(c + 2)
        pos_add(b)
        out_cp[c] = pltpu.async_copy(
            rows_bufs[b],
            out_hbm.at[pl.ds(base_row + c * ROWS_PER_CHUNK, ROWS_PER_CHUNK)],
            sem_o[b])
    out_cp[N_CHUNKS - 2].wait()
    out_cp[N_CHUNKS - 1].wait()


def kernel(x, token_table, pos_table):
    idx2d = x.reshape(-1).astype(jnp.int32)
    mesh = plsc.VectorSubcoreMesh(core_axis_name="c", subcore_axis_name="s")
    run = functools.partial(
        pl.kernel,
        mesh=mesh,
        out_type=jax.ShapeDtypeStruct((BATCH * MAXLEN, EMBED), jnp.float32),
        scratch_types=[
            pltpu.VMEM((ROWS_PER_CHUNK,), jnp.int32),
            pltpu.VMEM((ROWS_PER_CHUNK,), jnp.int32),
            pltpu.VMEM((ROWS_PER_CHUNK, EMBED), jnp.float32),
            pltpu.VMEM((ROWS_PER_CHUNK, EMBED), jnp.float32),
            pltpu.VMEM((MAXLEN, EMBED), jnp.float32),
            pltpu.SemaphoreType.DMA,
            pltpu.SemaphoreType.DMA,
            pltpu.SemaphoreType.DMA,
            pltpu.SemaphoreType.DMA,
            pltpu.SemaphoreType.DMA,
            pltpu.SemaphoreType.DMA,
        ],
        compiler_params=pltpu.CompilerParams(use_tc_tiling_on_sc=False),
    )(_embed_kernel)
    out = run(idx2d, token_table, pos_table)
    return out.reshape(BATCH, MAXLEN, EMBED)


# speedup vs baseline: 1.0011x; 1.0011x over previous
import functools

import jax
import jax.numpy as jnp
from jax import lax
from jax.experimental import pallas as pl
from jax.experimental.pallas import tpu as pltpu
from jax.experimental.pallas import tpu_sc as plsc

VOCAB = 1000000
MAXLEN = 200
EMBED = 16
BATCH = 4096

NUM_CORES = 2
NUM_SUBCORES = 16
NW = NUM_CORES * NUM_SUBCORES
SEQ_PER_W = BATCH // NW
CHUNK_SEQ = 16
ROWS_PER_CHUNK = CHUNK_SEQ * MAXLEN
GATHER = 128
N_GATHERS = ROWS_PER_CHUNK // GATHER
N_CHUNKS = SEQ_PER_W // CHUNK_SEQ
IDX_ROWS_PER_W = SEQ_PER_W * MAXLEN // GATHER


def _embed_kernel(x_hbm, tab_hbm, pos_hbm, out_hbm,
                  idx_a, idx_b, rows_a, rows_b, pos_v,
                  sem_i0, sem_i1, sem_g0, sem_g1, sem_o0, sem_o1):
    wid = lax.axis_index("s") * NUM_CORES + lax.axis_index("c")
    idx_bufs = (idx_a, idx_b)
    rows_bufs = (rows_a, rows_b)
    sem_i = (sem_i0, sem_i1)
    sem_g = (sem_g0, sem_g1)
    sem_o = (sem_o0, sem_o1)

    pltpu.sync_copy(pos_hbm, pos_v)
    base_row = wid * (SEQ_PER_W * MAXLEN)
    idx_base = wid * (SEQ_PER_W * MAXLEN)

    def fire_idx(c):
        b = c % 2
        return pltpu.async_copy(
            x_hbm.at[pl.ds(idx_base + c * ROWS_PER_CHUNK, ROWS_PER_CHUNK)],
            idx_bufs[b], sem_i[b])

    def fire_gathers(c):
        b = c % 2
        return [
            pltpu.async_copy(
                tab_hbm.at[idx_bufs[b]],
                rows_bufs[b],
                sem_g[b])
        ]

    def pos_add(b):
        def pos_body(p, carry):
            pv = pos_v[p, :]
            for s in range(CHUNK_SEQ):
                r = s * MAXLEN + p
                rows_bufs[b][r, :] = rows_bufs[b][r, :] + pv
            return carry

        lax.fori_loop(0, MAXLEN, pos_body, 0)

    idx_cp = {0: fire_idx(0)}
    idx_cp[0].wait()
    g_cps = {0: fire_gathers(0)}
    if N_CHUNKS > 1:
        idx_cp[1] = fire_idx(1)

    out_cp = {}
    for c in range(N_CHUNKS):
        b = c % 2
        if c + 1 < N_CHUNKS:
            if c - 1 in out_cp:
                out_cp[c - 1].wait()
            idx_cp[c + 1].wait()
            g_cps[c + 1] = fire_gathers(c + 1)
        for cp in g_cps[c]:
            cp.wait()
        if c + 2 < N_CHUNKS:
            idx_cp[c + 2] = fire_idx(c + 2)
        pos_add(b)
        out_cp[c] = pltpu.async_copy(
            rows_bufs[b],
            out_hbm.at[pl.ds(base_row + c * ROWS_PER_CHUNK, ROWS_PER_CHUNK)],
            sem_o[b])
    out_cp[N_CHUNKS - 2].wait()
    out_cp[N_CHUNKS - 1].wait()


def kernel(x, token_table, pos_table):
    idx2d = x.reshape(-1).astype(jnp.int32)
    mesh = plsc.VectorSubcoreMesh(core_axis_name="c", subcore_axis_name="s")
    run = functools.partial(
        pl.kernel,
        mesh=mesh,
        out_type=jax.ShapeDtypeStruct((BATCH * MAXLEN, EMBED), jnp.float32),
        scratch_types=[
            pltpu.VMEM((ROWS_PER_CHUNK,), jnp.int32),
            pltpu.VMEM((ROWS_PER_CHUNK,), jnp.int32),
            pltpu.VMEM((ROWS_PER_CHUNK, EMBED), jnp.float32),
            pltpu.VMEM((ROWS_PER_CHUNK, EMBED), jnp.float32),
            pltpu.VMEM((MAXLEN, EMBED), jnp.float32),
            pltpu.SemaphoreType.DMA,
            pltpu.SemaphoreType.DMA,
            pltpu.SemaphoreType.DMA,
            pltpu.SemaphoreType.DMA,
            pltpu.SemaphoreType.DMA,
            pltpu.SemaphoreType.DMA,
        ],
        compiler_params=pltpu.CompilerParams(use_tc_tiling_on_sc=False),
    )(_embed_kernel)
    out = run(idx2d, token_table, pos_table)
    return out.reshape(BATCH, MAXLEN, EMBED)
